# baseline (speedup 1.0000x reference)
import jax
import jax.numpy as jnp
from jax.experimental import pallas as pl
from jax.experimental.pallas import tpu as pltpu

_N = 100000
_D = 128
_C = 10
_BLK = 10000
_NB = _N // _BLK
_EPS = 1e-5
_SLOPE = 0.1


def _fused_kernel(x0_ref, ones_ref, w_ref, brel_ref,
                  gnw_ref, gnb_ref, gnms_ref, wlt_ref, blin_ref, out_ref,
                  conv_ref, s1_ref, s2_ref, carry_ref, acc_ref):
    p = pl.program_id(0)
    b = pl.program_id(1)
    rows = pl.ds(b * _BLK, _BLK)
    pm1 = jnp.maximum(p - 1, 0)
    bf16 = jnp.bfloat16
    dims = (((0,), (0,)), ((), ()))

    def norm_coeffs():
        m1 = s1_ref[pm1, 0:1, :] * (1.0 / _N)
        e2 = s2_ref[pm1, 0:1, :] * (1.0 / _N)
        br = brel_ref[0]
        mean = m1 + br
        ex2 = e2 + (2.0 * m1 + br) * br
        ms = gnms_ref[0]
        var = ex2 - ms * (2.0 - ms) * mean * mean
        alpha = gnw_ref[0] * jax.lax.rsqrt(var + _EPS)
        beta = gnb_ref[0] + alpha * (br - ms * mean)
        return alpha, beta

    def conv_tail(x):
        @pl.when(b == 0)
        def _():
            carry_ref[...] = jnp.zeros_like(carry_ref)
            s1_ref[p] = jnp.zeros_like(s1_ref[p])
            s2_ref[p] = jnp.zeros_like(s2_ref[p])

        xs = jnp.concatenate([carry_ref[...], x[:-1, :]], axis=0)
        carry_ref[...] = x[-1:, :]
        u = jnp.concatenate([xs, x], axis=1)
        c = jnp.dot(u, w_ref[0], preferred_element_type=jnp.float32)
        cb = c.astype(bf16)
        conv_ref[rows, :] = cb
        s1_ref[p] += jax.lax.dot_general(
            ones_ref[...], cb, dims, preferred_element_type=jnp.float32)
        s2_ref[p] += jax.lax.dot_general(
            ones_ref[...], cb * cb, dims,
            preferred_element_type=jnp.float32)

    @pl.when(p == 0)
    def _():
        conv_tail(x0_ref[...])

    @pl.when((p >= 1) & (p <= 3))
    def _():
        alpha, beta = norm_coeffs()
        y = conv_ref[rows, :] * alpha.astype(bf16) + beta.astype(bf16)
        conv_tail(jnp.maximum(y, bf16(_SLOPE) * y))

    @pl.when(p == 4)
    def _():
        alpha, beta = norm_coeffs()
        y = conv_ref[rows, :] * alpha.astype(bf16) + beta.astype(bf16)
        z = jnp.maximum(x0_ref[...] + y, bf16(0.0))

        @pl.when(b == 0)
        def _():
            acc_ref[...] = jnp.zeros_like(acc_ref)

        acc_ref[...] += jax.lax.dot_general(
            ones_ref[...], z, dims, preferred_element_type=jnp.float32)

        @pl.when(b == _NB - 1)
        def _():
            pooled = acc_ref[0:1, :] * (1.0 / _N)
            logits = jnp.dot(pooled, wlt_ref[...],
                             preferred_element_type=jnp.float32) + blin_ref[...]
            out_ref[...] = jax.nn.softplus(logits)


def kernel(feat, W_rel0, b_rel0, W_root0, gn_w0, gn_b0, gn_ms0,
           W_rel1, b_rel1, W_root1, gn_w1, gn_b1, gn_ms1,
           W_rel2, b_rel2, W_root2, gn_w2, gn_b2, gn_ms2,
           W_rel3, b_rel3, W_root3, gn_w3, gn_b3, gn_ms3,
           W_lin, b_lin):
    x0 = feat[0]
    f32 = jnp.float32
    bf16 = jnp.bfloat16

    w_all = jnp.stack([
        jnp.concatenate([Wr.T, Wo.T], axis=0)
        for Wr, Wo in ((W_rel0, W_root0), (W_rel1, W_root1),
                       (W_rel2, W_root2), (W_rel3, W_root3))]).astype(bf16)
    brel_all = jnp.stack([b_rel0, b_rel1, b_rel2, b_rel3])[:, None, :]
    gnw_all = jnp.stack([gn_w0, gn_w1, gn_w2, gn_w3])[:, None, :]
    gnb_all = jnp.stack([gn_b0, gn_b1, gn_b2, gn_b3])[:, None, :]
    gnms_all = jnp.stack([gn_ms0, gn_ms1, gn_ms2, gn_ms3])[:, None, :]
    ones_col = jnp.ones((_BLK, 8), bf16)

    def x0_map(p, b):
        return (jnp.where((p == 0) | (p == 4), b, 0), 0)

    def conv_w_map(p, b):
        return (jnp.minimum(p, 3), 0, 0)

    def gn_map(p, b):
        return (jnp.maximum(p - 1, 0), 0, 0)

    out = pl.pallas_call(
        _fused_kernel,
        grid=(5, _NB),
        in_specs=[
            pl.BlockSpec((_BLK, _D), x0_map),
            pl.BlockSpec((_BLK, 8), lambda p, b: (0, 0)),
            pl.BlockSpec((1, 2 * _D, _D), conv_w_map),
            pl.BlockSpec((1, 1, _D), gn_map),
            pl.BlockSpec((1, 1, _D), gn_map),
            pl.BlockSpec((1, 1, _D), gn_map),
            pl.BlockSpec((1, 1, _D), gn_map),
            pl.BlockSpec((_D, _C), lambda p, b: (0, 0)),
            pl.BlockSpec((1, _C), lambda p, b: (0, 0)),
        ],
        out_specs=pl.BlockSpec((1, _C), lambda p, b: (0, 0)),
        out_shape=jax.ShapeDtypeStruct((1, _C), f32),
        scratch_shapes=[
            pltpu.VMEM((_N, _D), bf16),
            pltpu.VMEM((4, 8, _D), f32),
            pltpu.VMEM((4, 8, _D), f32),
            pltpu.VMEM((1, _D), bf16),
            pltpu.VMEM((8, _D), f32),
        ],
    )(x0.astype(bf16), ones_col, w_all, brel_all, gnw_all, gnb_all,
      gnms_all, W_lin.T, b_lin.reshape(1, _C))

    return out.reshape(_C)


# speedup vs baseline: 1.1052x; 1.1052x over previous
import jax
import jax.numpy as jnp
from jax.experimental import pallas as pl
from jax.experimental.pallas import tpu as pltpu

_N = 100000
_D = 128
_C = 10
_BLK = 10000
_NB = _N // _BLK
_EPS = 1e-5
_SLOPE = 0.1


def _fused_kernel(x0_ref, ones_ref, w2_ref, brel_ref,
                  gnw_ref, gnb_ref, gnms_ref, wlt_ref, blin_ref, out_ref,
                  conv_ref, s1_ref, s2_ref, carry_ref, acc_ref):
    p = pl.program_id(0)
    b = pl.program_id(1)
    rows = pl.ds(b * _BLK, _BLK)
    pm1 = jnp.maximum(p - 1, 0)
    bf16 = jnp.bfloat16
    dims = (((0,), (0,)), ((), ()))

    def norm_coeffs():
        m1 = s1_ref[pm1, 0:1, :] * (1.0 / _N)
        e2 = s2_ref[pm1, 0:1, :] * (1.0 / _N)
        br = brel_ref[0]
        mean = m1 + br
        ex2 = e2 + (2.0 * m1 + br) * br
        ms = gnms_ref[0]
        var = ex2 - ms * (2.0 - ms) * mean * mean
        alpha = gnw_ref[0] * jax.lax.rsqrt(var + _EPS)
        beta = gnb_ref[0] + alpha * (br - ms * mean)
        return alpha, beta

    def conv_tail(xin):
        prod = jnp.dot(xin, w2_ref[0], preferred_element_type=jnp.float32)
        a = prod[:, :_D]
        bb = prod[:, _D:]

        @pl.when(b == 0)
        def _():
            carry_ref[...] = jnp.zeros_like(carry_ref)
            s1_ref[p] = jnp.zeros_like(s1_ref[p])
            s2_ref[p] = jnp.zeros_like(s2_ref[p])

        c = jnp.concatenate([carry_ref[...], a[:-1, :]], axis=0) + bb
        carry_ref[...] = a[-1:, :]
        cb = c.astype(bf16)
        conv_ref[rows, :] = cb
        s1_ref[p] += jax.lax.dot_general(
            ones_ref[...], cb, dims, preferred_element_type=jnp.float32)
        s2_ref[p] += jax.lax.dot_general(
            ones_ref[...], cb * cb, dims,
            preferred_element_type=jnp.float32)

    @pl.when(p == 0)
    def _():
        conv_tail(x0_ref[...])

    @pl.when((p >= 1) & (p <= 3))
    def _():
        alpha, beta = norm_coeffs()
        y = conv_ref[rows, :] * alpha.astype(bf16) + beta.astype(bf16)
        conv_tail(jnp.maximum(y, bf16(_SLOPE) * y))

    @pl.when(p == 4)
    def _():
        alpha, beta = norm_coeffs()
        y = conv_ref[rows, :] * alpha.astype(bf16) + beta.astype(bf16)
        z = jnp.maximum(x0_ref[...] + y, bf16(0.0))

        @pl.when(b == 0)
        def _():
            acc_ref[...] = jnp.zeros_like(acc_ref)

        acc_ref[...] += jax.lax.dot_general(
            ones_ref[...], z, dims, preferred_element_type=jnp.float32)

        @pl.when(b == _NB - 1)
        def _():
            pooled = acc_ref[0:1, :] * (1.0 / _N)
            logits = jnp.dot(pooled, wlt_ref[...],
                             preferred_element_type=jnp.float32) + blin_ref[...]
            out_ref[...] = jax.nn.softplus(logits)


def kernel(feat, W_rel0, b_rel0, W_root0, gn_w0, gn_b0, gn_ms0,
           W_rel1, b_rel1, W_root1, gn_w1, gn_b1, gn_ms1,
           W_rel2, b_rel2, W_root2, gn_w2, gn_b2, gn_ms2,
           W_rel3, b_rel3, W_root3, gn_w3, gn_b3, gn_ms3,
           W_lin, b_lin):
    x0 = feat[0]
    f32 = jnp.float32
    bf16 = jnp.bfloat16

    w2_all = jnp.stack([
        jnp.concatenate([Wr.T, Wo.T], axis=1)
        for Wr, Wo in ((W_rel0, W_root0), (W_rel1, W_root1),
                       (W_rel2, W_root2), (W_rel3, W_root3))]).astype(bf16)
    brel_all = jnp.stack([b_rel0, b_rel1, b_rel2, b_rel3])[:, None, :]
    gnw_all = jnp.stack([gn_w0, gn_w1, gn_w2, gn_w3])[:, None, :]
    gnb_all = jnp.stack([gn_b0, gn_b1, gn_b2, gn_b3])[:, None, :]
    gnms_all = jnp.stack([gn_ms0, gn_ms1, gn_ms2, gn_ms3])[:, None, :]
    ones_col = jnp.ones((_BLK, 8), bf16)

    def x0_map(p, b):
        return (jnp.where((p == 0) | (p == 4), b, 0), 0)

    def conv_w_map(p, b):
        return (jnp.minimum(p, 3), 0, 0)

    def gn_map(p, b):
        return (jnp.maximum(p - 1, 0), 0, 0)

    out = pl.pallas_call(
        _fused_kernel,
        grid=(5, _NB),
        in_specs=[
            pl.BlockSpec((_BLK, _D), x0_map),
            pl.BlockSpec((_BLK, 8), lambda p, b: (0, 0)),
            pl.BlockSpec((1, _D, 2 * _D), conv_w_map),
            pl.BlockSpec((1, 1, _D), gn_map),
            pl.BlockSpec((1, 1, _D), gn_map),
            pl.BlockSpec((1, 1, _D), gn_map),
            pl.BlockSpec((1, 1, _D), gn_map),
            pl.BlockSpec((_D, _C), lambda p, b: (0, 0)),
            pl.BlockSpec((1, _C), lambda p, b: (0, 0)),
        ],
        out_specs=pl.BlockSpec((1, _C), lambda p, b: (0, 0)),
        out_shape=jax.ShapeDtypeStruct((1, _C), f32),
        scratch_shapes=[
            pltpu.VMEM((_N, _D), bf16),
            pltpu.VMEM((4, 8, _D), f32),
            pltpu.VMEM((4, 8, _D), f32),
            pltpu.VMEM((1, _D), f32),
            pltpu.VMEM((8, _D), f32),
        ],
    )(x0.astype(bf16), ones_col, w2_all, brel_all, gnw_all, gnb_all,
      gnms_all, W_lin.T, b_lin.reshape(1, _C))

    return out.reshape(_C)


# speedup vs baseline: 1.1696x; 1.0583x over previous
import jax
import jax.numpy as jnp
from jax.experimental import pallas as pl
from jax.experimental.pallas import tpu as pltpu

_N = 100000
_D = 128
_C = 10
_BLK = 20000
_SUB = 10000
_NSUB = _BLK // _SUB
_NB = _N // _BLK
_EPS = 1e-5
_SLOPE = 0.1


def _fused_kernel(x0_ref, ones_ref, w2_ref, brel_ref,
                  gnw_ref, gnb_ref, gnms_ref, wlt_ref, blin_ref, out_ref,
                  conv_ref, s1_ref, s2_ref, carry_ref, acc_ref):
    p = pl.program_id(0)
    b = pl.program_id(1)
    pm1 = jnp.maximum(p - 1, 0)
    bf16 = jnp.bfloat16
    dims = (((0,), (0,)), ((), ()))

    def sub_rows(k):
        return pl.ds(b * _BLK + k * _SUB, _SUB)

    def norm_coeffs():
        m1 = s1_ref[pm1, 0:1, :] * (1.0 / _N)
        e2 = s2_ref[pm1, 0:1, :] * (1.0 / _N)
        br = brel_ref[0]
        mean = m1 + br
        ex2 = e2 + (2.0 * m1 + br) * br
        ms = gnms_ref[0]
        var = ex2 - ms * (2.0 - ms) * mean * mean
        alpha = gnw_ref[0] * jax.lax.rsqrt(var + _EPS)
        beta = gnb_ref[0] + alpha * (br - ms * mean)
        return alpha, beta

    def conv_tail(k, xin):
        prod = jnp.dot(xin, w2_ref[0], preferred_element_type=jnp.float32)
        a = prod[:, :_D]
        bb = prod[:, _D:]

        if k == 0:
            @pl.when(b == 0)
            def _():
                carry_ref[...] = jnp.zeros_like(carry_ref)
                s1_ref[p] = jnp.zeros_like(s1_ref[p])
                s2_ref[p] = jnp.zeros_like(s2_ref[p])

        c = jnp.concatenate([carry_ref[...], a[:-1, :]], axis=0) + bb
        carry_ref[...] = a[-1:, :]
        cb = c.astype(bf16)
        conv_ref[sub_rows(k), :] = cb
        s1_ref[p] += jax.lax.dot_general(
            ones_ref[...], cb, dims, preferred_element_type=jnp.float32)
        s2_ref[p] += jax.lax.dot_general(
            ones_ref[...], cb * cb, dims,
            preferred_element_type=jnp.float32)

    @pl.when(p == 0)
    def _():
        for k in range(_NSUB):
            conv_tail(k, x0_ref[k * _SUB:(k + 1) * _SUB, :])

    @pl.when((p >= 1) & (p <= 3))
    def _():
        alpha, beta = norm_coeffs()
        ab, bb16 = alpha.astype(bf16), beta.astype(bf16)
        for k in range(_NSUB):
            y = conv_ref[sub_rows(k), :] * ab + bb16
            conv_tail(k, jnp.maximum(y, bf16(_SLOPE) * y))

    @pl.when(p == 4)
    def _():
        alpha, beta = norm_coeffs()
        ab, bb16 = alpha.astype(bf16), beta.astype(bf16)

        @pl.when(b == 0)
        def _():
            acc_ref[...] = jnp.zeros_like(acc_ref)

        for k in range(_NSUB):
            y = conv_ref[sub_rows(k), :] * ab + bb16
            z = jnp.maximum(x0_ref[k * _SUB:(k + 1) * _SUB, :] + y, bf16(0.0))
            acc_ref[...] += jax.lax.dot_general(
                ones_ref[...], z, dims, preferred_element_type=jnp.float32)

        @pl.when(b == _NB - 1)
        def _():
            pooled = acc_ref[0:1, :] * (1.0 / _N)
            logits = jnp.dot(pooled, wlt_ref[...],
                             preferred_element_type=jnp.float32) + blin_ref[...]
            out_ref[...] = jax.nn.softplus(logits)


def kernel(feat, W_rel0, b_rel0, W_root0, gn_w0, gn_b0, gn_ms0,
           W_rel1, b_rel1, W_root1, gn_w1, gn_b1, gn_ms1,
           W_rel2, b_rel2, W_root2, gn_w2, gn_b2, gn_ms2,
           W_rel3, b_rel3, W_root3, gn_w3, gn_b3, gn_ms3,
           W_lin, b_lin):
    x0 = feat[0]
    f32 = jnp.float32
    bf16 = jnp.bfloat16

    w2_all = jnp.stack([
        jnp.concatenate([Wr.T, Wo.T], axis=1)
        for Wr, Wo in ((W_rel0, W_root0), (W_rel1, W_root1),
                       (W_rel2, W_root2), (W_rel3, W_root3))]).astype(bf16)
    brel_all = jnp.stack([b_rel0, b_rel1, b_rel2, b_rel3])[:, None, :]
    gnw_all = jnp.stack([gn_w0, gn_w1, gn_w2, gn_w3])[:, None, :]
    gnb_all = jnp.stack([gn_b0, gn_b1, gn_b2, gn_b3])[:, None, :]
    gnms_all = jnp.stack([gn_ms0, gn_ms1, gn_ms2, gn_ms3])[:, None, :]
    ones_col = jnp.ones((_SUB, 8), bf16)

    def x0_map(p, b):
        return (jnp.where((p == 0) | (p == 4), b, 0), 0)

    def conv_w_map(p, b):
        return (jnp.minimum(p, 3), 0, 0)

    def gn_map(p, b):
        return (jnp.maximum(p - 1, 0), 0, 0)

    out = pl.pallas_call(
        _fused_kernel,
        grid=(5, _NB),
        in_specs=[
            pl.BlockSpec((_BLK, _D), x0_map),
            pl.BlockSpec((_SUB, 8), lambda p, b: (0, 0)),
            pl.BlockSpec((1, _D, 2 * _D), conv_w_map),
            pl.BlockSpec((1, 1, _D), gn_map),
            pl.BlockSpec((1, 1, _D), gn_map),
            pl.BlockSpec((1, 1, _D), gn_map),
            pl.BlockSpec((1, 1, _D), gn_map),
            pl.BlockSpec((_D, _C), lambda p, b: (0, 0)),
            pl.BlockSpec((1, _C), lambda p, b: (0, 0)),
        ],
        out_specs=pl.BlockSpec((1, _C), lambda p, b: (0, 0)),
        out_shape=jax.ShapeDtypeStruct((1, _C), f32),
        scratch_shapes=[
            pltpu.VMEM((_N, _D), bf16),
            pltpu.VMEM((4, 8, _D), f32),
            pltpu.VMEM((4, 8, _D), f32),
            pltpu.VMEM((1, _D), f32),
            pltpu.VMEM((8, _D), f32),
        ],
    )(x0.astype(bf16), ones_col, w2_all, brel_all, gnw_all, gnb_all,
      gnms_all, W_lin.T, b_lin.reshape(1, _C))

    return out.reshape(_C)


# speedup vs baseline: 1.2033x; 1.0287x over previous
import jax
import jax.numpy as jnp
from jax.experimental import pallas as pl
from jax.experimental.pallas import tpu as pltpu

_N = 100000
_D = 128
_C = 10
_BLK = 20000
_SUB = 4000
_NSUB = _BLK // _SUB
_NB = _N // _BLK
_EPS = 1e-5
_SLOPE = 0.1


def _fused_kernel(x0_ref, ones_ref, w2_ref, brel_ref,
                  gnw_ref, gnb_ref, gnms_ref, wlt_ref, blin_ref, out_ref,
                  conv_ref, s1_ref, s2_ref, carry_ref, acc_ref):
    p = pl.program_id(0)
    b = pl.program_id(1)
    pm1 = jnp.maximum(p - 1, 0)
    bf16 = jnp.bfloat16
    dims = (((0,), (0,)), ((), ()))

    def sub_rows(k):
        return pl.ds(b * _BLK + k * _SUB, _SUB)

    def norm_coeffs():
        m1 = s1_ref[pm1, 0:1, :] * (1.0 / _N)
        e2 = s2_ref[pm1, 0:1, :] * (1.0 / _N)
        br = brel_ref[0]
        mean = m1 + br
        ex2 = e2 + (2.0 * m1 + br) * br
        ms = gnms_ref[0]
        var = ex2 - ms * (2.0 - ms) * mean * mean
        alpha = gnw_ref[0] * jax.lax.rsqrt(var + _EPS)
        beta = gnb_ref[0] + alpha * (br - ms * mean)
        return alpha, beta

    def conv_tail(k, xin):
        prod = jnp.dot(xin, w2_ref[0], preferred_element_type=jnp.float32)
        a = prod[:, :_D]
        bb = prod[:, _D:]

        if k == 0:
            @pl.when(b == 0)
            def _():
                carry_ref[...] = jnp.zeros_like(carry_ref)
                s1_ref[p] = jnp.zeros_like(s1_ref[p])
                s2_ref[p] = jnp.zeros_like(s2_ref[p])

        c = jnp.concatenate([carry_ref[...], a[:-1, :]], axis=0) + bb
        carry_ref[...] = a[-1:, :]
        cb = c.astype(bf16)
        conv_ref[sub_rows(k), :] = cb
        s1_ref[p] += jax.lax.dot_general(
            ones_ref[...], cb, dims, preferred_element_type=jnp.float32)
        s2_ref[p] += jax.lax.dot_general(
            ones_ref[...], cb * cb, dims,
            preferred_element_type=jnp.float32)

    @pl.when(p == 0)
    def _():
        for k in range(_NSUB):
            conv_tail(k, x0_ref[k * _SUB:(k + 1) * _SUB, :])

    @pl.when((p >= 1) & (p <= 3))
    def _():
        alpha, beta = norm_coeffs()
        ab, bb16 = alpha.astype(bf16), beta.astype(bf16)
        for k in range(_NSUB):
            y = conv_ref[sub_rows(k), :] * ab + bb16
            conv_tail(k, jnp.maximum(y, bf16(_SLOPE) * y))

    @pl.when(p == 4)
    def _():
        alpha, beta = norm_coeffs()
        ab, bb16 = alpha.astype(bf16), beta.astype(bf16)

        @pl.when(b == 0)
        def _():
            acc_ref[...] = jnp.zeros_like(acc_ref)

        for k in range(_NSUB):
            y = conv_ref[sub_rows(k), :] * ab + bb16
            z = jnp.maximum(x0_ref[k * _SUB:(k + 1) * _SUB, :] + y, bf16(0.0))
            acc_ref[...] += jax.lax.dot_general(
                ones_ref[...], z, dims, preferred_element_type=jnp.float32)

        @pl.when(b == _NB - 1)
        def _():
            pooled = acc_ref[0:1, :] * (1.0 / _N)
            logits = jnp.dot(pooled, wlt_ref[...],
                             preferred_element_type=jnp.float32) + blin_ref[...]
            out_ref[...] = jax.nn.softplus(logits)


def kernel(feat, W_rel0, b_rel0, W_root0, gn_w0, gn_b0, gn_ms0,
           W_rel1, b_rel1, W_root1, gn_w1, gn_b1, gn_ms1,
           W_rel2, b_rel2, W_root2, gn_w2, gn_b2, gn_ms2,
           W_rel3, b_rel3, W_root3, gn_w3, gn_b3, gn_ms3,
           W_lin, b_lin):
    x0 = feat[0]
    f32 = jnp.float32
    bf16 = jnp.bfloat16

    w2_all = jnp.stack([
        jnp.concatenate([Wr.T, Wo.T], axis=1)
        for Wr, Wo in ((W_rel0, W_root0), (W_rel1, W_root1),
                       (W_rel2, W_root2), (W_rel3, W_root3))]).astype(bf16)
    brel_all = jnp.stack([b_rel0, b_rel1, b_rel2, b_rel3])[:, None, :]
    gnw_all = jnp.stack([gn_w0, gn_w1, gn_w2, gn_w3])[:, None, :]
    gnb_all = jnp.stack([gn_b0, gn_b1, gn_b2, gn_b3])[:, None, :]
    gnms_all = jnp.stack([gn_ms0, gn_ms1, gn_ms2, gn_ms3])[:, None, :]
    ones_col = jnp.ones((_SUB, 8), bf16)

    def x0_map(p, b):
        return (jnp.where((p == 0) | (p == 4), b, 0), 0)

    def conv_w_map(p, b):
        return (jnp.minimum(p, 3), 0, 0)

    def gn_map(p, b):
        return (jnp.maximum(p - 1, 0), 0, 0)

    out = pl.pallas_call(
        _fused_kernel,
        grid=(5, _NB),
        in_specs=[
            pl.BlockSpec((_BLK, _D), x0_map),
            pl.BlockSpec((_SUB, 8), lambda p, b: (0, 0)),
            pl.BlockSpec((1, _D, 2 * _D), conv_w_map),
            pl.BlockSpec((1, 1, _D), gn_map),
            pl.BlockSpec((1, 1, _D), gn_map),
            pl.BlockSpec((1, 1, _D), gn_map),
            pl.BlockSpec((1, 1, _D), gn_map),
            pl.BlockSpec((_D, _C), lambda p, b: (0, 0)),
            pl.BlockSpec((1, _C), lambda p, b: (0, 0)),
        ],
        out_specs=pl.BlockSpec((1, _C), lambda p, b: (0, 0)),
        out_shape=jax.ShapeDtypeStruct((1, _C), f32),
        scratch_shapes=[
            pltpu.VMEM((_N, _D), bf16),
            pltpu.VMEM((4, 8, _D), f32),
            pltpu.VMEM((4, 8, _D), f32),
            pltpu.VMEM((1, _D), f32),
            pltpu.VMEM((8, _D), f32),
        ],
    )(x0.astype(bf16), ones_col, w2_all, brel_all, gnw_all, gnb_all,
      gnms_all, W_lin.T, b_lin.reshape(1, _C))

    return out.reshape(_C)
